# baseline (speedup 1.0000x reference)
import functools

import jax
import jax.numpy as jnp
from jax import lax
from jax.experimental import pallas as pl
from jax.experimental.pallas import tpu as pltpu
from jax.experimental.pallas import tpu_sc as plsc

N_NODES = 10000
N_PAD = 10240
D = 128
N_EDGES = 320000
NC, NS = 2, 16
NW = NC * NS
EDGES_PER_TILE = N_EDGES // NW
BATCH = 128
NBUF = 2
GB = 2
NBATCH = 80
EPT_PAD = NBATCH * BATCH
IDEPTH = 2 * NBUF
N_SH = 10240
ROWS_SH = N_SH // NS
ROWS_PER_TILE = N_PAD // NS
RB = 512
GRID = N_PAD // RB

_mesh = plsc.VectorSubcoreMesh(core_axis_name="c", subcore_axis_name="s")


@functools.partial(
    pl.kernel,
    out_type=jax.ShapeDtypeStruct((NC, N_PAD), jnp.float32),
    mesh=_mesh,
    scratch_types=[
        pltpu.VMEM((NBATCH, BATCH), jnp.int32),
        pltpu.VMEM((BATCH,), jnp.float32),
        pltpu.VMEM_SHARED((N_PAD,), jnp.float32),
        pltpu.SemaphoreType.DMA,
    ],
)
def _sc_degree(dst_hbm, ones_hbm, zeros_hbm, deg_hbm, dstv, onesv, shared, sem):
    c = lax.axis_index("c")
    s = lax.axis_index("s")
    wid = c * NS + s
    pltpu.sync_copy(dst_hbm.at[wid], dstv)
    pltpu.sync_copy(ones_hbm, onesv)
    pltpu.sync_copy(zeros_hbm.at[pl.ds(s * ROWS_PER_TILE, ROWS_PER_TILE)],
                    shared.at[pl.ds(s * ROWS_PER_TILE, ROWS_PER_TILE)])
    plsc.subcore_barrier()

    def body(j, carry):
        pltpu.sync_copy(onesv, shared.at[dstv.at[j]], add=True)
        return carry

    lax.fori_loop(0, NBATCH, body, 0)
    plsc.subcore_barrier()
    pltpu.sync_copy(shared.at[pl.ds(s * ROWS_PER_TILE, ROWS_PER_TILE)],
                    deg_hbm.at[c, pl.ds(s * ROWS_PER_TILE, ROWS_PER_TILE)])


@functools.partial(
    pl.kernel,
    out_type=jax.ShapeDtypeStruct((NC, N_PAD, D), jnp.float32),
    mesh=_mesh,
    scratch_types=[
        pltpu.VMEM((NBATCH, BATCH), jnp.int32),
        pltpu.VMEM((NBATCH, BATCH), jnp.int32),
        pltpu.VMEM((BATCH, D), jnp.float32),
        pltpu.VMEM_SHARED((N_SH, D), jnp.float32),
        pltpu.SemaphoreType.DMA,
    ],
)
def _sc_aggregate(hs_hbm, src_hbm, dst_hbm, zeros_hbm, agg_hbm,
                  srcv, dstv, rows, shared, gsem):
    c = lax.axis_index("c")
    s = lax.axis_index("s")
    wid = c * NS + s
    pltpu.sync_copy(src_hbm.at[wid], srcv)
    pltpu.sync_copy(dst_hbm.at[wid], dstv)
    pltpu.sync_copy(zeros_hbm, shared.at[pl.ds(s * ROWS_SH, ROWS_SH)])
    plsc.subcore_barrier()

    def body(j, carry):
        pltpu.async_copy(hs_hbm.at[srcv.at[j]], rows, gsem).wait()
        pltpu.sync_copy(rows, shared.at[dstv.at[j]], add=True)
        return carry

    lax.fori_loop(0, NBATCH, body, 0)

    plsc.subcore_barrier()
    pltpu.sync_copy(shared.at[pl.ds(s * ROWS_SH, ROWS_SH)],
                    agg_hbm.at[c, pl.ds(s * ROWS_SH, ROWS_SH)])


def _m1_body(x_ref, w_ref, deg_ref, hs_ref, dis_ref):
    d = deg_ref[...]
    dis = lax.rsqrt(d[0] + d[1] + 1.0)
    h = jnp.dot(x_ref[...], w_ref[...], preferred_element_type=jnp.float32)
    hs_ref[...] = h * dis
    dis_ref[...] = dis


def _m2_body(agg_ref, hs1_ref, dis_ref, b_ref, w_ref, out_ref):
    a = agg_ref[...]
    dis = dis_ref[...]
    agg = a[0] + a[1] + hs1_ref[...]
    o1 = jnp.maximum(agg * dis + b_ref[...], 0.0)
    h2 = jnp.dot(o1, w_ref[...], preferred_element_type=jnp.float32)
    row = lax.broadcasted_iota(jnp.int32, (RB, 1), 0) + pl.program_id(0) * RB
    keep = (row < N_NODES).astype(jnp.float32)
    out_ref[...] = h2 * dis * keep


def _m3_body(agg_ref, hs2_ref, dis_ref, b_ref, out_ref):
    a = agg_ref[...]
    out_ref[...] = (a[0] + a[1] + hs2_ref[...]) * dis_ref[...] + b_ref[...]


_m1 = pl.pallas_call(
    _m1_body,
    grid=(GRID,),
    in_specs=[
        pl.BlockSpec((RB, D), lambda i: (i, 0)),
        pl.BlockSpec((D, D), lambda i: (0, 0)),
        pl.BlockSpec((NC, RB, 1), lambda i: (0, i, 0)),
    ],
    out_specs=[
        pl.BlockSpec((RB, D), lambda i: (i, 0)),
        pl.BlockSpec((RB, 1), lambda i: (i, 0)),
    ],
    out_shape=[
        jax.ShapeDtypeStruct((N_PAD, D), jnp.float32),
        jax.ShapeDtypeStruct((N_PAD, 1), jnp.float32),
    ],
)

_m2 = pl.pallas_call(
    _m2_body,
    grid=(GRID,),
    in_specs=[
        pl.BlockSpec((NC, RB, D), lambda i: (0, i, 0)),
        pl.BlockSpec((RB, D), lambda i: (i, 0)),
        pl.BlockSpec((RB, 1), lambda i: (i, 0)),
        pl.BlockSpec((1, D), lambda i: (0, 0)),
        pl.BlockSpec((D, D), lambda i: (0, 0)),
    ],
    out_specs=pl.BlockSpec((RB, D), lambda i: (i, 0)),
    out_shape=jax.ShapeDtypeStruct((N_PAD, D), jnp.float32),
)

_m3 = pl.pallas_call(
    _m3_body,
    grid=(GRID,),
    in_specs=[
        pl.BlockSpec((NC, RB, D), lambda i: (0, i, 0)),
        pl.BlockSpec((RB, D), lambda i: (i, 0)),
        pl.BlockSpec((RB, 1), lambda i: (i, 0)),
        pl.BlockSpec((1, D), lambda i: (0, 0)),
    ],
    out_specs=pl.BlockSpec((RB, D), lambda i: (i, 0)),
    out_shape=jax.ShapeDtypeStruct((N_PAD, D), jnp.float32),
)


def kernel(x, edge_index, W1, b1, W2, b2):
    src = edge_index[0].astype(jnp.int32)
    dst = edge_index[1].astype(jnp.int32)
    src3 = jnp.pad(src.reshape(NW, EDGES_PER_TILE),
                   ((0, 0), (0, EPT_PAD - EDGES_PER_TILE)),
                   constant_values=N_NODES).reshape(NW, NBATCH, BATCH)
    dst3 = jnp.pad(dst.reshape(NW, EDGES_PER_TILE),
                   ((0, 0), (0, EPT_PAD - EDGES_PER_TILE)),
                   constant_values=N_NODES).reshape(NW, NBATCH, BATCH)

    x_pad = jnp.pad(x, ((0, N_PAD - N_NODES), (0, 0)))
    ones_b = jnp.ones((BATCH,), jnp.float32)
    zeros1 = jnp.zeros((N_PAD,), jnp.float32)
    zeros2 = jnp.zeros((ROWS_SH, D), jnp.float32)

    deg_p = _sc_degree(dst3, ones_b, zeros1)
    deg_r = deg_p.reshape(NC, N_PAD, 1)

    hs1, dis = _m1(x_pad, W1, deg_r)
    agg1 = _sc_aggregate(hs1, src3, dst3, zeros2)
    hs2 = _m2(agg1, hs1, dis, b1.reshape(1, D), W2)
    agg2 = _sc_aggregate(hs2, src3, dst3, zeros2)
    out = _m3(agg2, hs2, dis, b2.reshape(1, D))
    return out[:N_NODES]


# speedup vs baseline: 2.2185x; 2.2185x over previous
import functools

import jax
import jax.numpy as jnp
from jax import lax
from jax.experimental import pallas as pl
from jax.experimental.pallas import tpu as pltpu
from jax.experimental.pallas import tpu_sc as plsc

N_NODES = 10000
N_PAD = 10240
D = 128
N_EDGES = 320000
NC, NS = 2, 16
NW = NC * NS
EDGES_PER_TILE = N_EDGES // NW
BATCH = 128
NBATCH = 78
TAIL = EDGES_PER_TILE - NBATCH * BATCH
N_SH = 10240
ROWS_SH = N_SH // NS
ROWS_PER_TILE = N_PAD // NS
RB = 512
GRID = N_PAD // RB

_mesh = plsc.VectorSubcoreMesh(core_axis_name="c", subcore_axis_name="s")


@functools.partial(
    pl.kernel,
    out_type=jax.ShapeDtypeStruct((NC, N_PAD), jnp.float32),
    mesh=_mesh,
    scratch_types=[
        pltpu.VMEM((NBATCH, BATCH), jnp.int32),
        pltpu.VMEM((TAIL,), jnp.int32),
        pltpu.VMEM((BATCH,), jnp.float32),
        pltpu.VMEM_SHARED((N_PAD,), jnp.float32),
        pltpu.SemaphoreType.DMA,
    ],
)
def _sc_degree(dst_hbm, dstt_hbm, ones_hbm, zeros_hbm, deg_hbm,
               dstv, dstt, onesv, shared, sem):
    c = lax.axis_index("c")
    s = lax.axis_index("s")
    wid = c * NS + s
    pltpu.sync_copy(dst_hbm.at[wid], dstv)
    pltpu.sync_copy(dstt_hbm.at[wid], dstt)
    pltpu.sync_copy(ones_hbm, onesv)
    pltpu.sync_copy(zeros_hbm.at[pl.ds(s * ROWS_PER_TILE, ROWS_PER_TILE)],
                    shared.at[pl.ds(s * ROWS_PER_TILE, ROWS_PER_TILE)])
    plsc.subcore_barrier()

    def body(j, carry):
        pltpu.sync_copy(onesv, shared.at[dstv.at[j]], add=True)
        return carry

    lax.fori_loop(0, NBATCH, body, 0)
    pltpu.sync_copy(onesv.at[pl.ds(0, TAIL)], shared.at[dstt], add=True)
    plsc.subcore_barrier()
    pltpu.sync_copy(shared.at[pl.ds(s * ROWS_PER_TILE, ROWS_PER_TILE)],
                    deg_hbm.at[c, pl.ds(s * ROWS_PER_TILE, ROWS_PER_TILE)])


@functools.partial(
    pl.kernel,
    out_type=jax.ShapeDtypeStruct((NC, N_PAD, D), jnp.float32),
    mesh=_mesh,
    scratch_types=[
        pltpu.VMEM((NBATCH, BATCH), jnp.int32),
        pltpu.VMEM((NBATCH, BATCH), jnp.int32),
        pltpu.VMEM((TAIL,), jnp.int32),
        pltpu.VMEM((TAIL,), jnp.int32),
        pltpu.VMEM((BATCH, D), jnp.float32),
        pltpu.VMEM_SHARED((N_SH, D), jnp.float32),
        pltpu.SemaphoreType.DMA,
    ],
)
def _sc_aggregate(hs_hbm, src_hbm, dst_hbm, srct_hbm, dstt_hbm,
                  zeros_hbm, agg_hbm,
                  srcv, dstv, srct, dstt, rows, shared, gsem):
    c = lax.axis_index("c")
    s = lax.axis_index("s")
    wid = c * NS + s
    pltpu.sync_copy(src_hbm.at[wid], srcv)
    pltpu.sync_copy(dst_hbm.at[wid], dstv)
    pltpu.sync_copy(srct_hbm.at[wid], srct)
    pltpu.sync_copy(dstt_hbm.at[wid], dstt)
    pltpu.sync_copy(zeros_hbm, shared.at[pl.ds(s * ROWS_SH, ROWS_SH)])
    plsc.subcore_barrier()

    def body(j, carry):
        pltpu.async_copy(hs_hbm.at[srcv.at[j]], rows, gsem).wait()
        pltpu.sync_copy(rows, shared.at[dstv.at[j]], add=True)
        return carry

    lax.fori_loop(0, NBATCH, body, 0)
    pltpu.async_copy(hs_hbm.at[srct], rows.at[pl.ds(0, TAIL)], gsem).wait()
    pltpu.sync_copy(rows.at[pl.ds(0, TAIL)], shared.at[dstt], add=True)

    plsc.subcore_barrier()
    pltpu.sync_copy(shared.at[pl.ds(s * ROWS_SH, ROWS_SH)],
                    agg_hbm.at[c, pl.ds(s * ROWS_SH, ROWS_SH)])


def _m1_body(x_ref, w_ref, deg_ref, hs_ref, dis_ref):
    d = deg_ref[...]
    dis = lax.rsqrt(d[0] + d[1] + 1.0)
    h = jnp.dot(x_ref[...], w_ref[...], preferred_element_type=jnp.float32)
    hs_ref[...] = h * dis
    dis_ref[...] = dis


def _m2_body(agg_ref, hs1_ref, dis_ref, b_ref, w_ref, out_ref):
    a = agg_ref[...]
    dis = dis_ref[...]
    agg = a[0] + a[1] + hs1_ref[...]
    o1 = jnp.maximum(agg * dis + b_ref[...], 0.0)
    h2 = jnp.dot(o1, w_ref[...], preferred_element_type=jnp.float32)
    row = lax.broadcasted_iota(jnp.int32, (RB, 1), 0) + pl.program_id(0) * RB
    keep = (row < N_NODES).astype(jnp.float32)
    out_ref[...] = h2 * dis * keep


def _m3_body(agg_ref, hs2_ref, dis_ref, b_ref, out_ref):
    a = agg_ref[...]
    out_ref[...] = (a[0] + a[1] + hs2_ref[...]) * dis_ref[...] + b_ref[...]


_m1 = pl.pallas_call(
    _m1_body,
    grid=(GRID,),
    in_specs=[
        pl.BlockSpec((RB, D), lambda i: (i, 0)),
        pl.BlockSpec((D, D), lambda i: (0, 0)),
        pl.BlockSpec((NC, RB, 1), lambda i: (0, i, 0)),
    ],
    out_specs=[
        pl.BlockSpec((RB, D), lambda i: (i, 0)),
        pl.BlockSpec((RB, 1), lambda i: (i, 0)),
    ],
    out_shape=[
        jax.ShapeDtypeStruct((N_PAD, D), jnp.float32),
        jax.ShapeDtypeStruct((N_PAD, 1), jnp.float32),
    ],
)

_m2 = pl.pallas_call(
    _m2_body,
    grid=(GRID,),
    in_specs=[
        pl.BlockSpec((NC, RB, D), lambda i: (0, i, 0)),
        pl.BlockSpec((RB, D), lambda i: (i, 0)),
        pl.BlockSpec((RB, 1), lambda i: (i, 0)),
        pl.BlockSpec((1, D), lambda i: (0, 0)),
        pl.BlockSpec((D, D), lambda i: (0, 0)),
    ],
    out_specs=pl.BlockSpec((RB, D), lambda i: (i, 0)),
    out_shape=jax.ShapeDtypeStruct((N_PAD, D), jnp.float32),
)

_m3 = pl.pallas_call(
    _m3_body,
    grid=(GRID,),
    in_specs=[
        pl.BlockSpec((NC, RB, D), lambda i: (0, i, 0)),
        pl.BlockSpec((RB, D), lambda i: (i, 0)),
        pl.BlockSpec((RB, 1), lambda i: (i, 0)),
        pl.BlockSpec((1, D), lambda i: (0, 0)),
    ],
    out_specs=pl.BlockSpec((RB, D), lambda i: (i, 0)),
    out_shape=jax.ShapeDtypeStruct((N_PAD, D), jnp.float32),
)


def kernel(x, edge_index, W1, b1, W2, b2):
    src = edge_index[0].astype(jnp.int32)
    dst = edge_index[1].astype(jnp.int32)
    src2 = src.reshape(NW, EDGES_PER_TILE)
    dst2 = dst.reshape(NW, EDGES_PER_TILE)
    src3 = src2[:, :NBATCH * BATCH].reshape(NW, NBATCH, BATCH)
    dst3 = dst2[:, :NBATCH * BATCH].reshape(NW, NBATCH, BATCH)
    src_t = src2[:, NBATCH * BATCH:]
    dst_t = dst2[:, NBATCH * BATCH:]

    x_pad = jnp.pad(x, ((0, N_PAD - N_NODES), (0, 0)))
    ones_b = jnp.ones((BATCH,), jnp.float32)
    zeros1 = jnp.zeros((N_PAD,), jnp.float32)
    zeros2 = jnp.zeros((ROWS_SH, D), jnp.float32)

    deg_p = _sc_degree(dst3, dst_t, ones_b, zeros1)
    deg_r = deg_p.reshape(NC, N_PAD, 1)

    hs1, dis = _m1(x_pad, W1, deg_r)
    agg1 = _sc_aggregate(hs1, src3, dst3, src_t, dst_t, zeros2)
    hs2 = _m2(agg1, hs1, dis, b1.reshape(1, D), W2)
    agg2 = _sc_aggregate(hs2, src3, dst3, src_t, dst_t, zeros2)
    out = _m3(agg2, hs2, dis, b2.reshape(1, D))
    return out[:N_NODES]
